# baseline (speedup 1.0000x reference)
import functools

import jax
import jax.numpy as jnp
from jax import lax
from jax.experimental import pallas as pl
from jax.experimental.pallas import tpu as pltpu
from jax.experimental.pallas import tpu_sc as plsc


def _gather_kernel(seq, batch, d, dpad, n_workers, nc):
    blk = batch // n_workers
    cs = 2
    n_steps = seq // cs
    assert n_steps % 2 == 0
    mesh = plsc.VectorSubcoreMesh(core_axis_name="c", subcore_axis_name="s")

    @functools.partial(
        pl.kernel,
        mesh=mesh,
        compiler_params=pltpu.CompilerParams(needs_layout_passes=False),
        out_type=jax.ShapeDtypeStruct((seq, d, batch), jnp.float32),
        scratch_types=[
            pltpu.VMEM((cs, blk), jnp.int32),
            pltpu.VMEM((cs, blk), jnp.int32),
            pltpu.VMEM((cs * blk, dpad), jnp.float32),
            pltpu.VMEM((cs * blk, dpad), jnp.float32),
            pltpu.VMEM((cs, d, blk), jnp.float32),
            pltpu.VMEM((cs, d, blk), jnp.float32),
            pltpu.SemaphoreType.DMA,
            pltpu.SemaphoreType.DMA,
            pltpu.SemaphoreType.DMA,
            pltpu.SemaphoreType.DMA,
            pltpu.SemaphoreType.DMA,
            pltpu.SemaphoreType.DMA,
        ],
    )
    def k(table_hbm, xt_hbm, out_hbm, ix0, ix1, buf0, buf1, tr0, tr1,
          i0, i1, g0, g1, s0, s1):
        wid = lax.axis_index("s") * nc + lax.axis_index("c")
        b0 = wid * blk
        lane = lax.iota(jnp.int32, 16)

        def wait_gather(buf, gsem):
            for u in range(cs):
                pltpu.make_async_copy(
                    table_hbm.at[ix0.at[u]],
                    buf.at[pl.ds(u * blk, blk)],
                    gsem,
                ).wait()

        def store(t, tr, sem):
            pltpu.make_async_copy(
                tr, out_hbm.at[pl.ds(t * cs, cs), :, pl.ds(b0, blk)], sem
            ).start()

        def wait_store(tr, sem):
            pltpu.make_async_copy(
                tr, out_hbm.at[pl.ds(0, cs), :, pl.ds(b0, blk)], sem
            ).wait()

        def transpose(buf, tr):
            @plsc.parallel_loop(0, d, 1, unroll=2)
            def _(c):
                cols = jnp.full((16,), 0, jnp.int32) + c
                vals = []
                for u in range(cs):
                    for kk in range(blk // 16):
                        rows = lane + (u * blk + 16 * kk)
                        vals.append(plsc.load_gather(buf, [rows, cols]))
                i = 0
                for u in range(cs):
                    for kk in range(blk // 16):
                        tr[u, c, pl.ds(16 * kk, 16)] = vals[i]
                        i += 1

        def start_fetch(t, ix, isem, buf, gsem):
            pltpu.make_async_copy(
                xt_hbm.at[pl.ds(t * cs, cs), pl.ds(b0, blk)], ix, isem
            ).start()

        def finish_fetch(ix, isem, buf, gsem):
            pltpu.make_async_copy(
                xt_hbm.at[pl.ds(0, cs), pl.ds(b0, blk)], ix, isem
            ).wait()
            for u in range(cs):
                pltpu.make_async_copy(
                    table_hbm.at[ix.at[u]],
                    buf.at[pl.ds(u * blk, blk)],
                    gsem,
                ).start()

        start_fetch(0, ix0, i0, buf0, g0)
        finish_fetch(ix0, i0, buf0, g0)
        start_fetch(1, ix1, i1, buf1, g1)

        def body(j, carry):
            ta = 2 * j
            tb = 2 * j + 1

            finish_fetch(ix1, i1, buf1, g1)

            wait_gather(buf0, g0)

            @pl.when(j > 0)
            def _():
                wait_store(tr0, s0)

            transpose(buf0, tr0)
            store(ta, tr0, s0)

            @pl.when(j < n_steps // 2 - 1)
            def _():
                start_fetch(ta + 2, ix0, i0, buf0, g0)
                finish_fetch(ix0, i0, buf0, g0)

            wait_gather(buf1, g1)

            @pl.when(j > 0)
            def _():
                wait_store(tr1, s1)

            transpose(buf1, tr1)
            store(tb, tr1, s1)

            @pl.when(j < n_steps // 2 - 1)
            def _():
                start_fetch(tb + 2, ix1, i1, buf1, g1)

            return carry

        lax.fori_loop(0, n_steps // 2, body, 0)
        wait_store(tr0, s0)
        wait_store(tr1, s1)

    return k


def kernel(x, table):
    b, s = x.shape
    v, d = table.shape

    info = plsc.get_sparse_core_info()
    nc, ns = info.num_cores, info.num_subcores
    n_workers = nc * ns

    dpad = 128
    filler = jnp.broadcast_to(table[:1, :], (v, dpad - d))
    table_w = jnp.concatenate([table, filler], axis=1)
    xt = x.T.astype(jnp.int32)
    out_t = _gather_kernel(s, b, d, dpad, n_workers, nc)(table_w, xt)
    return jnp.transpose(out_t, (2, 0, 1))


# speedup vs baseline: 1.2225x; 1.2225x over previous
import functools

import jax
import jax.numpy as jnp
from jax import lax
from jax.experimental import pallas as pl
from jax.experimental.pallas import tpu as pltpu
from jax.experimental.pallas import tpu_sc as plsc


def _gather_kernel(seq, batch, d, dpad, n_workers, nc):
    blk = batch // n_workers
    cs = 2
    n_steps = seq // cs
    assert n_steps % 2 == 0
    mesh = plsc.VectorSubcoreMesh(core_axis_name="c", subcore_axis_name="s")

    @functools.partial(
        pl.kernel,
        mesh=mesh,
        compiler_params=pltpu.CompilerParams(needs_layout_passes=False),
        out_type=jax.ShapeDtypeStruct((seq, d, batch), jnp.float32),
        scratch_types=[
            pltpu.VMEM((cs, blk), jnp.int32),
            pltpu.VMEM((cs, blk), jnp.int32),
            pltpu.VMEM((cs * blk, dpad), jnp.float32),
            pltpu.VMEM((cs * blk, dpad), jnp.float32),
            pltpu.VMEM((cs, d, blk), jnp.float32),
            pltpu.VMEM((cs, d, blk), jnp.float32),
            pltpu.SemaphoreType.DMA,
            pltpu.SemaphoreType.DMA,
            pltpu.SemaphoreType.DMA,
            pltpu.SemaphoreType.DMA,
            pltpu.SemaphoreType.DMA,
            pltpu.SemaphoreType.DMA,
        ],
    )
    def k(table_hbm, xt_hbm, out_hbm, ix0, ix1, buf0, buf1, tr0, tr1,
          i0, i1, g0, g1, s0, s1):
        wid = lax.axis_index("s") * nc + lax.axis_index("c")
        b0 = wid * blk
        lane = lax.iota(jnp.int32, 16)

        def wait_gather(buf, gsem):
            for u in range(cs):
                pltpu.make_async_copy(
                    table_hbm.at[ix0.at[u]],
                    buf.at[pl.ds(u * blk, blk)],
                    gsem,
                ).wait()

        def store(t, tr, sem):
            pltpu.make_async_copy(
                tr, out_hbm.at[pl.ds(t * cs, cs), :, pl.ds(b0, blk)], sem
            ).start()

        def wait_store(tr, sem):
            pltpu.make_async_copy(
                tr, out_hbm.at[pl.ds(0, cs), :, pl.ds(b0, blk)], sem
            ).wait()

        def transpose(buf, tr):
            @plsc.parallel_loop(0, d, 1, unroll=4)
            def _(c):
                cols = jnp.full((16,), 0, jnp.int32) + c
                for u in range(cs):
                    for kk in range(blk // 16):
                        rows = lane + (u * blk + 16 * kk)
                        tr[u, c, pl.ds(16 * kk, 16)] = plsc.load_gather(
                            buf, [rows, cols]
                        )

        def start_fetch(t, ix, isem, buf, gsem):
            pltpu.make_async_copy(
                xt_hbm.at[pl.ds(t * cs, cs), pl.ds(b0, blk)], ix, isem
            ).start()

        def finish_fetch(ix, isem, buf, gsem):
            pltpu.make_async_copy(
                xt_hbm.at[pl.ds(0, cs), pl.ds(b0, blk)], ix, isem
            ).wait()
            for u in range(cs):
                pltpu.make_async_copy(
                    table_hbm.at[ix.at[u]],
                    buf.at[pl.ds(u * blk, blk)],
                    gsem,
                ).start()

        start_fetch(0, ix0, i0, buf0, g0)
        finish_fetch(ix0, i0, buf0, g0)
        start_fetch(1, ix1, i1, buf1, g1)

        def body(j, carry):
            ta = 2 * j
            tb = 2 * j + 1

            finish_fetch(ix1, i1, buf1, g1)

            wait_gather(buf0, g0)

            @pl.when(j > 0)
            def _():
                wait_store(tr0, s0)

            transpose(buf0, tr0)
            store(ta, tr0, s0)

            @pl.when(j < n_steps // 2 - 1)
            def _():
                start_fetch(ta + 2, ix0, i0, buf0, g0)
                finish_fetch(ix0, i0, buf0, g0)

            wait_gather(buf1, g1)

            @pl.when(j > 0)
            def _():
                wait_store(tr1, s1)

            transpose(buf1, tr1)
            store(tb, tr1, s1)

            @pl.when(j < n_steps // 2 - 1)
            def _():
                start_fetch(tb + 2, ix1, i1, buf1, g1)

            return carry

        lax.fori_loop(0, n_steps // 2, body, 0)
        wait_store(tr0, s0)
        wait_store(tr1, s1)

    return k


def kernel(x, table):
    b, s = x.shape
    v, d = table.shape

    info = plsc.get_sparse_core_info()
    nc, ns = info.num_cores, info.num_subcores
    n_workers = nc * ns

    dpad = 128
    filler = jnp.broadcast_to(table[:1, :], (v, dpad - d))
    table_w = jnp.concatenate([table, filler], axis=1)
    xt = x.T.astype(jnp.int32)
    out_t = _gather_kernel(s, b, d, dpad, n_workers, nc)(table_w, xt)
    return jnp.transpose(out_t, (2, 0, 1))


# speedup vs baseline: 1.2270x; 1.0037x over previous
import functools

import jax
import jax.numpy as jnp
from jax import lax
from jax.experimental import pallas as pl
from jax.experimental.pallas import tpu as pltpu
from jax.experimental.pallas import tpu_sc as plsc


def _gather_kernel(n_total, d, chunk, n_workers, nc):
    n_per_w = n_total // n_workers
    n_chunks = n_per_w // chunk
    assert n_chunks % 2 == 0
    n_outer = n_chunks // 2
    mesh = plsc.VectorSubcoreMesh(core_axis_name="c", subcore_axis_name="s")

    @functools.partial(
        pl.kernel,
        mesh=mesh,
        compiler_params=pltpu.CompilerParams(use_tc_tiling_on_sc=False),
        out_type=jax.ShapeDtypeStruct((n_total, d), jnp.float32),
        scratch_types=[
            pltpu.VMEM((n_per_w,), jnp.int32),
            pltpu.VMEM((chunk, d), jnp.float32),
            pltpu.VMEM((chunk, d), jnp.float32),
            pltpu.SemaphoreType.DMA,
            pltpu.SemaphoreType.DMA,
            pltpu.SemaphoreType.DMA,
            pltpu.SemaphoreType.DMA,
        ],
    )
    def k(table_hbm, idx_hbm, out_hbm, idx_v, rows0, rows1, g0, g1, s0, s1):
        wid = lax.axis_index("s") * nc + lax.axis_index("c")
        base = wid * n_per_w

        pltpu.sync_copy(idx_hbm.at[pl.ds(base, n_per_w)], idx_v)

        def gather(c, rows, sem):
            pltpu.make_async_copy(
                table_hbm.at[idx_v.at[pl.ds(c * chunk, chunk)]], rows, sem
            ).start()

        def store(c, rows, sem):
            pltpu.make_async_copy(
                rows, out_hbm.at[pl.ds(base + c * chunk, chunk)], sem
            ).start()

        def wait_g(rows, sem):
            pltpu.make_async_copy(table_hbm.at[idx_v.at[pl.ds(0, chunk)]], rows, sem).wait()

        def wait_s(rows, sem):
            pltpu.make_async_copy(rows, out_hbm.at[pl.ds(base, chunk)], sem).wait()

        gather(0, rows0, g0)

        def body(j, carry):
            @pl.when(j > 0)
            def _():
                wait_s(rows1, s1)

            gather(2 * j + 1, rows1, g1)

            wait_g(rows0, g0)
            store(2 * j, rows0, s0)

            @pl.when(j < n_outer - 1)
            def _():
                wait_s(rows0, s0)
                gather(2 * j + 2, rows0, g0)

            wait_g(rows1, g1)
            store(2 * j + 1, rows1, s1)
            return carry

        lax.fori_loop(0, n_outer, body, 0)
        wait_s(rows0, s0)
        wait_s(rows1, s1)

    return k


def kernel(x, table):
    b, s = x.shape
    v, d = table.shape
    n_total = b * s

    info = plsc.get_sparse_core_info()
    nc, ns = info.num_cores, info.num_subcores
    n_workers = nc * ns

    chunk = 512
    flat_idx = x.reshape(n_total).astype(jnp.int32)
    out = _gather_kernel(n_total, d, chunk, n_workers, nc)(table, flat_idx)
    return out.reshape(b, s, d)
